# baseline (speedup 1.0000x reference)
import functools

import jax
import jax.numpy as jnp
from jax import lax
from jax.experimental import pallas as pl
from jax.experimental.pallas import tpu as pltpu
from jax.experimental.pallas import tpu_sc as plsc

_GR = 16


@functools.lru_cache(maxsize=None)
def _make(n_rows: int, s: int, d: int):
    info = plsc.get_sparse_core_info()
    nc, ns = info.num_cores, info.num_subcores
    nw = nc * ns
    rows_w = n_rows // nw
    n_rounds = rows_w // _GR
    n_super = n_rounds // 2
    mesh = plsc.VectorSubcoreMesh(core_axis_name="c", subcore_axis_name="s")

    @functools.partial(
        pl.kernel,
        mesh=mesh,
        out_type=jax.ShapeDtypeStruct((n_rows, s, d), jnp.float32),
        scratch_types=[
            pltpu.VMEM((rows_w, s), jnp.int32),
            pltpu.VMEM((2, _GR, s, d), jnp.float32),
            pltpu.SemaphoreType.DMA,
            pltpu.SemaphoreType.DMA,
            pltpu.SemaphoreType.DMA,
            pltpu.SemaphoreType.DMA,
        ],
        compiler_params=pltpu.CompilerParams(use_tc_tiling_on_sc=False),
    )
    def gather_kernel(table_hbm, idx_hbm, out_hbm, idx_v, rows_v,
                      sem_g0, sem_g1, sem_w0, sem_w1):
        wid = lax.axis_index("s") * nc + lax.axis_index("c")
        r0_w = wid * rows_w
        pltpu.sync_copy(idx_hbm.at[pl.ds(r0_w, rows_w)], idx_v)

        sem_g = (sem_g0, sem_g1)
        sem_w = (sem_w0, sem_w1)

        def fire_gathers(r, g):
            for rr in range(_GR):
                pltpu.async_copy(
                    table_hbm.at[idx_v.at[r * _GR + rr]],
                    rows_v.at[g, rr], sem_g[g])

        def drain_gathers(g):
            for rr in range(_GR):
                pltpu.make_async_copy(
                    table_hbm.at[idx_v.at[0]],
                    rows_v.at[g, rr], sem_g[g]).wait()

        def fire_write(r, g):
            pltpu.async_copy(
                rows_v.at[g],
                out_hbm.at[pl.ds(r0_w + r * _GR, _GR)], sem_w[g])

        def drain_write(g):
            pltpu.make_async_copy(
                rows_v.at[g],
                out_hbm.at[pl.ds(0, _GR)], sem_w[g]).wait()

        fire_gathers(0, 0)

        def body(t, carry):
            r0 = 2 * t
            drain_gathers(0)
            fire_write(r0, 0)

            @pl.when(t > 0)
            def _():
                drain_write(1)
            fire_gathers(r0 + 1, 1)

            drain_gathers(1)
            fire_write(r0 + 1, 1)
            drain_write(0)

            @pl.when(t < n_super - 1)
            def _():
                fire_gathers(r0 + 2, 0)
            return carry

        lax.fori_loop(0, n_super, body, 0)
        drain_write(1)

    return gather_kernel


def kernel(x, weight):
    b, s = x.shape
    d = weight.shape[1]
    return _make(b, s, d)(weight, x.astype(jnp.int32))
